# Initial kernel scaffold; baseline (speedup 1.0000x reference)
import jax
import jax.numpy as jnp
from jax.experimental import pallas as pl


def kernel(x, edge_index, edge_attr, batch, node_table, W_sgc, b_sgc, W1, b1, g1, be1, W2, b2, g2, be2, W3, b3):
    raise NotImplementedError("write your pallas kernel here")


# speedup vs baseline: 18.4045x; 18.4045x over previous
import functools

import jax
import jax.numpy as jnp
from jax import lax
from jax.experimental import pallas as pl
from jax.experimental.pallas import tpu as pltpu
from jax.experimental.pallas import tpu_sc as plsc

N = 10000
E = 320000
D = 128
G = 64
VOCAB = 119
K = 4

NC = 2
NS = 16
NW = NC * NS
C = 128
CH = 80
E_PAD = NW * CH * C
N_PAD = 10240
RW = N_PAD // NS
XCH = N_PAD // C
GE = 8
GROUPS = CH // GE
PPG = GE // 2

_f32 = jnp.float32


def _zero_fill_2d(buf, rows):
    zv = jnp.zeros((16,), _f32)

    def body(i, _):
        for jj in range(D // 16):
            buf[i, pl.ds(jj * 16, 16)] = zv
        return 0

    lax.fori_loop(0, rows, body, 0)


@functools.lru_cache(maxsize=1)
def _build_sc_prep():
    mesh = plsc.VectorSubcoreMesh(core_axis_name="c", subcore_axis_name="s")

    @functools.partial(
        pl.kernel,
        out_type=[
            jax.ShapeDtypeStruct((NC * N_PAD,), _f32),
            jax.ShapeDtypeStruct((N_PAD, D), _f32),
        ],
        mesh=mesh,
        scratch_types=[
            pltpu.VMEM((CH, C), jnp.int32),
            pltpu.VMEM((C,), jnp.int32),
            pltpu.VMEM((C, D), _f32),
            pltpu.VMEM((C,), _f32),
            pltpu.VMEM((RW,), _f32),
            pltpu.VMEM_SHARED((N_PAD,), _f32),
            pltpu.SemaphoreType.DMA,
        ],
    )
    def sc_prep(cols_hbm, x_hbm, table_hbm, degp_hbm, h0_hbm,
                colv, xv, embuf, onesv, stage, degacc, sem):
        cid = lax.axis_index("c")
        sid = lax.axis_index("s")
        wid = cid * NS + sid

        def emb_chunk(cj):
            pltpu.sync_copy(x_hbm.at[cj], xv)
            pltpu.async_copy(table_hbm.at[xv], embuf, sem).wait()
            pltpu.sync_copy(embuf, h0_hbm.at[pl.ds(cj * C, C)])

        emb_chunk(wid)
        emb_chunk(wid + NW)

        @pl.when(wid < XCH - 2 * NW)
        def _():
            emb_chunk(wid + 2 * NW)

        def fill(i, _):
            stage[pl.ds(i * 16, 16)] = jnp.zeros((16,), _f32)
            return 0

        lax.fori_loop(0, RW // 16, fill, 0)
        for jj in range(C // 16):
            onesv[pl.ds(jj * 16, 16)] = jnp.ones((16,), _f32)

        pltpu.sync_copy(cols_hbm.at[wid], colv)
        base = sid * RW
        pltpu.sync_copy(stage, degacc.at[pl.ds(base, RW)])
        plsc.subcore_barrier()

        def scat(j, _):
            pltpu.sync_copy(onesv, degacc.at[colv.at[j]], add=True)
            return 0

        lax.fori_loop(0, CH, scat, 0)
        plsc.subcore_barrier()

        pltpu.sync_copy(degacc.at[pl.ds(base, RW)], stage)
        pltpu.sync_copy(stage, degp_hbm.at[pl.ds(cid * N_PAD + base, RW)])

    return sc_prep


@functools.lru_cache(maxsize=1)
def _build_sc_hop():
    mesh = plsc.VectorSubcoreMesh(core_axis_name="c", subcore_axis_name="s")

    @functools.partial(
        pl.kernel,
        out_type=jax.ShapeDtypeStruct((NC * N_PAD, D), _f32),
        mesh=mesh,
        scratch_types=[
            pltpu.VMEM((2, GE, C), jnp.int32),
            pltpu.VMEM((2, GE, C), jnp.int32),
            pltpu.VMEM((C, D), _f32),
            pltpu.VMEM((C, D), _f32),
            pltpu.VMEM_SHARED((N_PAD, D), _f32),
            pltpu.SemaphoreType.DMA,
            pltpu.SemaphoreType.DMA,
        ],
    )
    def sc_hop(g_hbm, rows_hbm, cols_hbm, u_hbm,
               rowv, colv, bufa, bufb, acc, sema, semb):
        cid = lax.axis_index("c")
        sid = lax.axis_index("s")
        wid = cid * NS + sid

        _zero_fill_2d(bufa, C)
        base = sid * RW
        for t in range(RW // C):
            pltpu.sync_copy(bufa, acc.at[pl.ds(base + t * C, C)])

        pltpu.sync_copy(rows_hbm.at[wid, pl.ds(0, GE)], rowv.at[0])
        pltpu.sync_copy(cols_hbm.at[wid, pl.ds(0, GE)], colv.at[0])
        plsc.subcore_barrier()

        def body(p, _):
            grp = p // PPG
            rp = lax.rem(p, PPG)
            par = lax.rem(grp, 2)
            r0 = 2 * rp
            pltpu.async_copy(g_hbm.at[rowv.at[par, r0]], bufa, sema)
            pltpu.async_copy(g_hbm.at[rowv.at[par, r0 + 1]], bufb, semb)

            @pl.when(jnp.logical_and(rp == 0, grp + 1 < GROUPS))
            def _():
                nxt = lax.rem(grp + 1, 2)
                pltpu.sync_copy(rows_hbm.at[wid, pl.ds((grp + 1) * GE, GE)],
                                rowv.at[nxt])
                pltpu.sync_copy(cols_hbm.at[wid, pl.ds((grp + 1) * GE, GE)],
                                colv.at[nxt])

            pltpu.make_async_copy(g_hbm.at[rowv.at[par, r0]], bufa, sema).wait()
            pltpu.sync_copy(bufa, acc.at[colv.at[par, r0]], add=True)
            pltpu.make_async_copy(
                g_hbm.at[rowv.at[par, r0 + 1]], bufb, semb).wait()
            pltpu.sync_copy(bufb, acc.at[colv.at[par, r0 + 1]], add=True)
            return 0

        lax.fori_loop(0, CH // 2, body, 0)
        plsc.subcore_barrier()

        for t in range(RW // C):
            pltpu.sync_copy(acc.at[pl.ds(base + t * C, C)], bufa)
            pltpu.sync_copy(
                bufa, u_hbm.at[pl.ds(cid * N_PAD + base + t * C, C)])

    return sc_hop


def _tc_scalings(degp_ref, h0_ref, g_ref, dinv_ref, s_ref):
    dp = degp_ref[...]
    deg = dp[:N_PAD] + dp[N_PAD:] + 1.0
    dinv = lax.rsqrt(deg)
    dinv_ref[...] = dinv
    s_ref[...] = 1.0 / deg
    g_ref[...] = h0_ref[...] * dinv


def _tc_combine(u_ref, g_ref, s_ref, out_ref):
    uu = u_ref[...]
    out_ref[...] = (uu[:N_PAD] + uu[N_PAD:] + g_ref[...]) * s_ref[...]


def _tc_head(u_ref, g_ref, dinv_ref, batch_ref, wsgc_ref, bsgc_ref,
             w1_ref, b1_ref, g1_ref, be1_ref, w2_ref, b2_ref, g2_ref, be2_ref,
             w3_ref, b3_ref, out_ref):
    def mm(a, b):
        return jnp.dot(a, b, precision=jax.lax.Precision.DEFAULT,
                       preferred_element_type=_f32)

    uu = u_ref[...]
    v = uu[:N_PAD] + uu[N_PAD:] + g_ref[...]
    h = v * dinv_ref[...]
    hw = mm(h, wsgc_ref[...]) + bsgc_ref[...]
    gid = lax.broadcasted_iota(jnp.int32, (G, 1), 0)
    oh = (batch_ref[...] == gid).astype(_f32)
    hg = jnp.dot(oh, hw, precision=jax.lax.Precision.HIGHEST,
                 preferred_element_type=_f32)

    def bn_relu(hh, gamma, beta):
        m = jnp.mean(hh, axis=0, keepdims=True)
        var = jnp.mean((hh - m) ** 2, axis=0, keepdims=True)
        hh = (hh - m) / jnp.sqrt(var + 1e-5) * gamma + beta
        return jnp.maximum(hh, 0.0)

    hg = bn_relu(mm(hg, w1_ref[...]) + b1_ref[...], g1_ref[...], be1_ref[...])
    hg = bn_relu(mm(hg, w2_ref[...]) + b2_ref[...], g2_ref[...], be2_ref[...])
    out_ref[...] = mm(hg, w3_ref[...]) + b3_ref[...]


def kernel(x, edge_index, edge_attr, batch, node_table, W_sgc, b_sgc,
           W1, b1, g1, be1, W2, b2, g2, be2, W3, b3):
    del edge_attr
    i32 = jnp.int32

    pad_e = E_PAD - E
    rows = edge_index[0].astype(i32)
    cols = edge_index[1].astype(i32)
    pad_tgt = N + (jnp.arange(pad_e, dtype=i32) % (N_PAD - N))
    rows_p = jnp.concatenate([rows, pad_tgt]).reshape(NW, CH, C)
    cols_p = jnp.concatenate([cols, pad_tgt]).reshape(NW, CH, C)

    x_p = jnp.concatenate(
        [x.astype(i32), jnp.full((N_PAD - N,), VOCAB, i32)]).reshape(XCH, C)
    table_p = jnp.zeros((C, D), _f32).at[:VOCAB].set(node_table)

    batch_p = jnp.concatenate(
        [batch.astype(i32), jnp.full((N_PAD - N,), G, i32)]).reshape(1, N_PAD)

    degp, h0 = _build_sc_prep()(cols_p, x_p, table_p)
    degp = degp.reshape(NC * N_PAD, 1)

    g, dinv, s = pl.pallas_call(
        _tc_scalings,
        out_shape=[
            jax.ShapeDtypeStruct((N_PAD, D), _f32),
            jax.ShapeDtypeStruct((N_PAD, 1), _f32),
            jax.ShapeDtypeStruct((N_PAD, 1), _f32),
        ],
    )(degp, h0)

    sc_hop = _build_sc_hop()
    combine = pl.pallas_call(
        _tc_combine, out_shape=jax.ShapeDtypeStruct((N_PAD, D), _f32))
    for _ in range(K - 1):
        u = sc_hop(g, rows_p, cols_p)
        g = combine(u, g, s)
    u = sc_hop(g, rows_p, cols_p)

    out = pl.pallas_call(
        _tc_head, out_shape=jax.ShapeDtypeStruct((G, 1), _f32),
    )(u, g, dinv, batch_p,
      W_sgc, b_sgc.reshape(1, D),
      W1, b1.reshape(1, 2 * D), g1.reshape(1, 2 * D), be1.reshape(1, 2 * D),
      W2, b2.reshape(1, D), g2.reshape(1, D), be2.reshape(1, D),
      W3, b3.reshape(1, 1))
    return out
